# baseline (speedup 1.0000x reference)
import functools

import jax
import jax.numpy as jnp
from jax import lax
from jax.experimental import pallas as pl
from jax.experimental.pallas import tpu as pltpu
from jax.experimental.pallas import tpu_sc as plsc

NC = 2
NS = 16
LANES = 16


def _sc_mean_agg(h2, src, dst, n_src, n_dst, fh, chunk, sb, interleave=False):
    E = src.shape[0]
    e_tile = E // NS
    n_chunks = e_tile // chunk
    n_sb = n_chunks // sb
    stripe = -(-n_dst // NS)
    stripe += (-stripe) % 8
    n_pad = stripe * NS
    nbuf = 4 if sb % 4 == 0 else 2
    la = nbuf // 2
    col_export = (2 * fh == 128)
    assert e_tile % chunk == 0 and n_chunks % sb == 0
    assert sb % nbuf == 0 and chunk % 8 == 0 and chunk <= 128

    src3 = src.astype(jnp.int32).reshape(NS, n_chunks, chunk)
    dst3 = dst.astype(jnp.int32).reshape(NS, n_chunks, chunk)

    mesh = plsc.VectorSubcoreMesh(core_axis_name="c", subcore_axis_name="s",
                                  num_cores=NC, num_subcores=NS)
    if col_export:
        agg_type = jax.ShapeDtypeStruct((n_pad, 2 * fh), jnp.float32)
        zeros_f = jnp.zeros((n_pad, fh), jnp.float32)
    else:
        agg_type = jax.ShapeDtypeStruct((NC, n_pad, fh), jnp.float32)
        zeros_f = jnp.zeros((n_pad, fh), jnp.float32)
    zeros_c = jnp.zeros((n_pad, 16), jnp.float32)
    ones_c = jnp.ones((chunk, 16), jnp.float32)

    @functools.partial(
        pl.kernel,
        out_type=(agg_type,
                  jax.ShapeDtypeStruct((NC, n_pad, 16), jnp.float32)),
        mesh=mesh,
        scratch_types=[
            pltpu.VMEM((sb, chunk), jnp.int32),
            pltpu.VMEM((sb, chunk), jnp.int32),
            pltpu.VMEM((chunk, fh), jnp.float32),
            pltpu.VMEM((chunk, fh), jnp.float32),
            pltpu.VMEM((chunk, fh), jnp.float32),
            pltpu.VMEM((chunk, fh), jnp.float32),
            pltpu.VMEM((chunk, 16), jnp.float32),
            pltpu.VMEM_SHARED((n_pad, fh), jnp.float32),
            pltpu.VMEM_SHARED((n_pad, 16), jnp.float32),
            pltpu.SemaphoreType.DMA,
            pltpu.SemaphoreType.DMA,
            pltpu.SemaphoreType.DMA,
            pltpu.SemaphoreType.DMA,
            pltpu.SemaphoreType.DMA,
            pltpu.SemaphoreType.DMA,
            pltpu.SemaphoreType.DMA,
            pltpu.SemaphoreType.DMA,
            pltpu.SemaphoreType.DMA,
        ],
        compiler_params=pltpu.CompilerParams(use_tc_tiling_on_sc=False),
    )
    def k(h2_hbm, src_hbm, dst_hbm, zf_hbm, zc_hbm, ones_hbm,
          agg_out, cnt_out,
          blk_s, blk_d, r0v, r1v, r2v, r3v, ones_v, acc_sh, cnt_sh,
          g0, g1, g2, g3, s0, s1, s2, s3, sem_c):
        rows = [r0v, r1v, r2v, r3v]
        gsem = [g0, g1, g2, g3]
        ssem = [s0, s1, s2, s3]
        cid = lax.axis_index("c")
        sid = lax.axis_index("s")
        r0 = sid * stripe
        zf_cp = pltpu.async_copy(zf_hbm.at[pl.ds(r0, stripe)],
                                 acc_sh.at[pl.ds(r0, stripe)], g0)
        zc_cp = pltpu.async_copy(zc_hbm.at[pl.ds(r0, stripe)],
                                 cnt_sh.at[pl.ds(r0, stripe)], g1)
        pltpu.sync_copy(ones_hbm, ones_v)
        zf_cp.wait()
        zc_cp.wait()
        plsc.subcore_barrier()

        base = cid if interleave else cid * n_src
        half = sb // 2

        def g_fire(j, b):
            return pltpu.async_copy(h2_hbm.at[blk_s.at[j]], rows[b], gsem[b])

        def g_wait(j, b):
            pltpu.make_async_copy(h2_hbm.at[blk_s.at[j]],
                                  rows[b], gsem[b]).wait()

        def s_fire(j, b):
            pltpu.async_copy(rows[b], acc_sh.at[blk_d.at[j]],
                             ssem[b], add=True)

        def s_wait(j, b):
            pltpu.make_async_copy(rows[b], acc_sh.at[blk_d.at[j]],
                                  ssem[b]).wait()

        def sb_body(s, carry):
            s_cp = pltpu.async_copy(src_hbm.at[sid, pl.ds(s * sb, sb)],
                                    blk_s, g0)
            pltpu.async_copy(dst_hbm.at[sid, pl.ds(s * sb, sb)],
                             blk_d, g1).wait()
            s_cp.wait()

            def tbody(j, c2):
                for i in range(chunk // LANES):
                    sl = pl.ds(i * LANES, LANES)
                    if interleave:
                        blk_s[j, sl] = blk_s[j, sl] * 2 + base
                    else:
                        blk_s[j, sl] = blk_s[j, sl] + base
                return c2

            lax.fori_loop(0, sb, tbody, 0)
            for j in range(la):
                g_fire(j, j % nbuf)

            def fbody(p, c2):
                j0 = p * nbuf
                for off in range(nbuf):
                    j = j0 + off
                    b = off
                    g_wait(j, b)

                    @pl.when((j < half) == (cid == 0))
                    def _():
                        pltpu.async_copy(ones_v, cnt_sh.at[blk_d.at[j]],
                                         sem_c, add=True)

                    s_fire(j, b)
                    bn = (off + la) % nbuf
                    jn = j + la

                    @pl.when(j >= nbuf - la)
                    def _():
                        s_wait(jn - nbuf, bn)

                    @pl.when(jn < sb)
                    def _():
                        g_fire(jn, bn)
                return c2

            lax.fori_loop(0, sb // nbuf, fbody, 0)
            for off in range(nbuf - la):
                j = sb - 1 - off
                s_wait(j, j % nbuf)

            def dbody(j, c2):
                pltpu.make_async_copy(ones_hbm, ones_v, sem_c).wait()
                return c2

            lax.fori_loop(0, half, dbody, 0)
            return carry

        lax.fori_loop(0, n_sb, sb_body, 0)
        plsc.subcore_barrier()
        if col_export:
            pltpu.sync_copy(acc_sh.at[pl.ds(r0, stripe)],
                            agg_out.at[pl.ds(r0, stripe),
                                       pl.ds(cid * fh, fh)])
        else:
            pltpu.sync_copy(acc_sh.at[pl.ds(r0, stripe)],
                            agg_out.at[cid, pl.ds(r0, stripe)])
        pltpu.sync_copy(cnt_sh.at[pl.ds(r0, stripe)],
                        cnt_out.at[cid, pl.ds(r0, stripe)])

    return k(h2, src3, dst3, zeros_f, zeros_c, ones_c)


def _tc_sage(h_prev, agg, cnt, w_self, w_neigh, b, relu, split_out, n_out, bm):
    stacked_in = h_prev.ndim == 3
    stacked_agg = agg.ndim == 3
    H = w_neigh.shape[-1]
    Hh = H // 2

    def body(hd_ref, a_ref, c_ref, ws_ref, wn_ref, b_ref, o_ref):
        c = jnp.maximum(c_ref[0, :, 0:1] + c_ref[1, :, 0:1], 1.0)
        if stacked_agg:
            neigh = jnp.dot(a_ref[0], wn_ref[0],
                            preferred_element_type=jnp.float32)
            neigh += jnp.dot(a_ref[1], wn_ref[1],
                             preferred_element_type=jnp.float32)
        else:
            neigh = jnp.dot(a_ref[...], wn_ref[...],
                            preferred_element_type=jnp.float32)
        if stacked_in:
            out = jnp.dot(hd_ref[0], ws_ref[0],
                          preferred_element_type=jnp.float32)
            out += jnp.dot(hd_ref[1], ws_ref[1],
                           preferred_element_type=jnp.float32)
        else:
            out = jnp.dot(hd_ref[...], ws_ref[...],
                          preferred_element_type=jnp.float32)
        out += neigh / c
        out += b_ref[...]
        if relu:
            out = jnp.maximum(out, 0.0)
        if split_out:
            o_ref[0] = out[:, :Hh]
            o_ref[1] = out[:, Hh:]
        else:
            o_ref[...] = out

    if stacked_in:
        hd_spec = pl.BlockSpec((2, bm, h_prev.shape[2]), lambda i: (0, i, 0))
        ws_spec = pl.BlockSpec(w_self.shape, lambda i: (0, 0, 0))
    else:
        hd_spec = pl.BlockSpec((bm, h_prev.shape[1]), lambda i: (i, 0))
        ws_spec = pl.BlockSpec(w_self.shape, lambda i: (0, 0))
    if stacked_agg:
        a_spec = pl.BlockSpec((2, bm, agg.shape[2]), lambda i: (0, i, 0))
        wn_spec = pl.BlockSpec(w_neigh.shape, lambda i: (0, 0, 0))
    else:
        a_spec = pl.BlockSpec((bm, agg.shape[1]), lambda i: (i, 0))
        wn_spec = pl.BlockSpec(w_neigh.shape, lambda i: (0, 0))
    if split_out:
        out_spec = pl.BlockSpec((2, bm, Hh), lambda i: (0, i, 0))
        out_shape = jax.ShapeDtypeStruct((2, n_out, Hh), jnp.float32)
    else:
        out_spec = pl.BlockSpec((bm, H), lambda i: (i, 0))
        out_shape = jax.ShapeDtypeStruct((n_out, H), jnp.float32)

    return pl.pallas_call(
        body,
        grid=(n_out // bm,),
        in_specs=[
            hd_spec,
            a_spec,
            pl.BlockSpec((2, bm, 16), lambda i: (0, i, 0)),
            ws_spec,
            wn_spec,
            pl.BlockSpec((1, H), lambda i: (0, 0)),
        ],
        out_specs=out_spec,
        out_shape=out_shape,
    )(h_prev, agg, cnt, w_self, w_neigh, b)


def kernel(x, src1, dst1, src2, dst2, src3, dst3, n_dst1, n_dst2, n_dst3,
           W_self1, W_neigh1, b1, W_self2, W_neigh2, b2,
           W_self3, W_neigh3, b3):
    N1, N2, N3 = 20000, 8000, 4096
    n_src = x.shape[0]
    zero = ((jnp.asarray(n_dst1) - N1)
            + (jnp.asarray(n_dst2) - N2)
            + (jnp.asarray(n_dst3) - N3)).astype(x.dtype)

    agg1, cnt1 = _sc_mean_agg(x.reshape(2 * n_src, 64), src1, dst1, n_src, N1,
                              fh=64, chunk=80, sb=50, interleave=True)
    h1 = _tc_sage(x, agg1, cnt1, W_self1, W_neigh1,
                  b1.reshape(1, 256), relu=True, split_out=True,
                  n_out=N1, bm=800)

    agg2, cnt2 = _sc_mean_agg(h1.reshape(2 * N1, 128), src2, dst2, N1, N2,
                              fh=128, chunk=80, sb=100)
    h2 = _tc_sage(h1, agg2, cnt2, W_self2.reshape(2, 128, 256),
                  W_neigh2.reshape(2, 128, 256), b2.reshape(1, 256),
                  relu=True, split_out=True, n_out=N2, bm=800)

    agg3, cnt3 = _sc_mean_agg(h2.reshape(2 * N2, 128), src3, dst3, N2, N3,
                              fh=128, chunk=128, sb=32)
    h3 = _tc_sage(h2, agg3, cnt3, W_self3.reshape(2, 128, 256),
                  W_neigh3.reshape(2, 128, 256), (b3 + zero).reshape(1, 256),
                  relu=False, split_out=False, n_out=N3, bm=1024)
    return h3


# speedup vs baseline: 1.1657x; 1.1657x over previous
import functools

import jax
import jax.numpy as jnp
from jax import lax
from jax.experimental import pallas as pl
from jax.experimental.pallas import tpu as pltpu
from jax.experimental.pallas import tpu_sc as plsc

NC = 2
NS = 16
LANES = 16


def _sc_mean_agg(h2, src, dst, n_src, n_dst, fh, chunk, sb, interleave=False):
    E = src.shape[0]
    e_tile = E // NS
    n_chunks = e_tile // chunk
    n_sb = n_chunks // sb
    stripe = -(-n_dst // NS)
    stripe += (-stripe) % 8
    n_pad = stripe * NS
    col_export = (2 * fh == 128)
    assert e_tile % chunk == 0 and n_chunks % sb == 0
    assert sb % 2 == 0 and chunk % 8 == 0 and chunk <= 128

    src3 = src.astype(jnp.int32).reshape(NS, n_chunks, chunk)
    dst3 = dst.astype(jnp.int32).reshape(NS, n_chunks, chunk)

    mesh = plsc.VectorSubcoreMesh(core_axis_name="c", subcore_axis_name="s",
                                  num_cores=NC, num_subcores=NS)
    if col_export:
        agg_type = jax.ShapeDtypeStruct((n_pad, 2 * fh), jnp.float32)
        zeros_f = jnp.zeros((n_pad, fh), jnp.float32)
    else:
        agg_type = jax.ShapeDtypeStruct((NC, n_pad, fh), jnp.float32)
        zeros_f = jnp.zeros((n_pad, fh), jnp.float32)
    zeros_c = jnp.zeros((n_pad, 16), jnp.float32)
    ones_c = jnp.ones((chunk, 16), jnp.float32)

    @functools.partial(
        pl.kernel,
        out_type=(agg_type,
                  jax.ShapeDtypeStruct((NC, n_pad, 16), jnp.float32)),
        mesh=mesh,
        scratch_types=[
            pltpu.VMEM((sb, chunk), jnp.int32),
            pltpu.VMEM((sb, chunk), jnp.int32),
            pltpu.VMEM((chunk, fh), jnp.float32),
            pltpu.VMEM((chunk, fh), jnp.float32),
            pltpu.VMEM((chunk, 16), jnp.float32),
            pltpu.VMEM_SHARED((n_pad, fh), jnp.float32),
            pltpu.VMEM_SHARED((n_pad, 16), jnp.float32),
            pltpu.SemaphoreType.DMA,
            pltpu.SemaphoreType.DMA,
            pltpu.SemaphoreType.DMA,
        ],
        compiler_params=pltpu.CompilerParams(use_tc_tiling_on_sc=False),
    )
    def k(h2_hbm, src_hbm, dst_hbm, zf_hbm, zc_hbm, ones_hbm,
          agg_out, cnt_out,
          blk_s, blk_d, r0v, r1v, ones_v, acc_sh, cnt_sh,
          g0, g1, sem_c):
        rows = [r0v, r1v]
        gsem = [g0, g1]
        cid = lax.axis_index("c")
        sid = lax.axis_index("s")
        r0 = sid * stripe
        zf_cp = pltpu.async_copy(zf_hbm.at[pl.ds(r0, stripe)],
                                 acc_sh.at[pl.ds(r0, stripe)], g0)
        zc_cp = pltpu.async_copy(zc_hbm.at[pl.ds(r0, stripe)],
                                 cnt_sh.at[pl.ds(r0, stripe)], g1)
        pltpu.sync_copy(ones_hbm, ones_v)
        zf_cp.wait()
        zc_cp.wait()
        plsc.subcore_barrier()

        base = cid if interleave else cid * n_src
        half = sb // 2

        def g_fire(j, b):
            return pltpu.async_copy(h2_hbm.at[blk_s.at[j]], rows[b], gsem[b])

        def g_wait(j, b):
            pltpu.make_async_copy(h2_hbm.at[blk_s.at[j]],
                                  rows[b], gsem[b]).wait()

        def sb_body(s, carry):
            s_cp = pltpu.async_copy(src_hbm.at[sid, pl.ds(s * sb, sb)],
                                    blk_s, g0)
            pltpu.async_copy(dst_hbm.at[sid, pl.ds(s * sb, sb)],
                             blk_d, g1).wait()
            s_cp.wait()

            def tbody(j, c2):
                for i in range(chunk // LANES):
                    sl = pl.ds(i * LANES, LANES)
                    if interleave:
                        blk_s[j, sl] = blk_s[j, sl] * 2 + base
                    else:
                        blk_s[j, sl] = blk_s[j, sl] + base
                return c2

            lax.fori_loop(0, sb, tbody, 0)
            g_fire(0, 0)

            def fbody(p, c2):
                j0 = 2 * p
                g_fire(j0 + 1, 1)

                @pl.when((j0 < half) == (cid == 0))
                def _():
                    pltpu.async_copy(ones_v, cnt_sh.at[blk_d.at[j0]],
                                     sem_c, add=True)

                g_wait(j0, 0)
                pltpu.sync_copy(rows[0], acc_sh.at[blk_d.at[j0]], add=True)

                @pl.when(j0 + 2 < sb)
                def _():
                    g_fire(j0 + 2, 0)

                @pl.when((j0 + 1 < half) == (cid == 0))
                def _():
                    pltpu.async_copy(ones_v, cnt_sh.at[blk_d.at[j0 + 1]],
                                     sem_c, add=True)

                g_wait(j0 + 1, 1)
                pltpu.sync_copy(rows[1], acc_sh.at[blk_d.at[j0 + 1]], add=True)
                return c2

            lax.fori_loop(0, sb // 2, fbody, 0)

            def dbody(j, c2):
                pltpu.make_async_copy(ones_hbm, ones_v, sem_c).wait()
                return c2

            lax.fori_loop(0, half, dbody, 0)
            return carry

        lax.fori_loop(0, n_sb, sb_body, 0)
        plsc.subcore_barrier()
        if col_export:
            pltpu.sync_copy(acc_sh.at[pl.ds(r0, stripe)],
                            agg_out.at[pl.ds(r0, stripe),
                                       pl.ds(cid * fh, fh)])
        else:
            pltpu.sync_copy(acc_sh.at[pl.ds(r0, stripe)],
                            agg_out.at[cid, pl.ds(r0, stripe)])
        pltpu.sync_copy(cnt_sh.at[pl.ds(r0, stripe)],
                        cnt_out.at[cid, pl.ds(r0, stripe)])

    return k(h2, src3, dst3, zeros_f, zeros_c, ones_c)


def _tc_sage(h_prev, agg, cnt, w_self, w_neigh, b, relu, split_out, n_out, bm):
    stacked_in = h_prev.ndim == 3
    stacked_agg = agg.ndim == 3
    H = w_neigh.shape[-1]
    Hh = H // 2

    def body(hd_ref, a_ref, c_ref, ws_ref, wn_ref, b_ref, o_ref):
        c = jnp.maximum(c_ref[0, :, 0:1] + c_ref[1, :, 0:1], 1.0)
        if stacked_agg:
            neigh = jnp.dot(a_ref[0], wn_ref[0],
                            preferred_element_type=jnp.float32)
            neigh += jnp.dot(a_ref[1], wn_ref[1],
                             preferred_element_type=jnp.float32)
        else:
            neigh = jnp.dot(a_ref[...], wn_ref[...],
                            preferred_element_type=jnp.float32)
        if stacked_in:
            out = jnp.dot(hd_ref[0], ws_ref[0],
                          preferred_element_type=jnp.float32)
            out += jnp.dot(hd_ref[1], ws_ref[1],
                           preferred_element_type=jnp.float32)
        else:
            out = jnp.dot(hd_ref[...], ws_ref[...],
                          preferred_element_type=jnp.float32)
        out += neigh / c
        out += b_ref[...]
        if relu:
            out = jnp.maximum(out, 0.0)
        if split_out:
            o_ref[0] = out[:, :Hh]
            o_ref[1] = out[:, Hh:]
        else:
            o_ref[...] = out

    if stacked_in:
        hd_spec = pl.BlockSpec((2, bm, h_prev.shape[2]), lambda i: (0, i, 0))
        ws_spec = pl.BlockSpec(w_self.shape, lambda i: (0, 0, 0))
    else:
        hd_spec = pl.BlockSpec((bm, h_prev.shape[1]), lambda i: (i, 0))
        ws_spec = pl.BlockSpec(w_self.shape, lambda i: (0, 0))
    if stacked_agg:
        a_spec = pl.BlockSpec((2, bm, agg.shape[2]), lambda i: (0, i, 0))
        wn_spec = pl.BlockSpec(w_neigh.shape, lambda i: (0, 0, 0))
    else:
        a_spec = pl.BlockSpec((bm, agg.shape[1]), lambda i: (i, 0))
        wn_spec = pl.BlockSpec(w_neigh.shape, lambda i: (0, 0))
    if split_out:
        out_spec = pl.BlockSpec((2, bm, Hh), lambda i: (0, i, 0))
        out_shape = jax.ShapeDtypeStruct((2, n_out, Hh), jnp.float32)
    else:
        out_spec = pl.BlockSpec((bm, H), lambda i: (i, 0))
        out_shape = jax.ShapeDtypeStruct((n_out, H), jnp.float32)

    return pl.pallas_call(
        body,
        grid=(n_out // bm,),
        in_specs=[
            hd_spec,
            a_spec,
            pl.BlockSpec((2, bm, 16), lambda i: (0, i, 0)),
            ws_spec,
            wn_spec,
            pl.BlockSpec((1, H), lambda i: (0, 0)),
        ],
        out_specs=out_spec,
        out_shape=out_shape,
    )(h_prev, agg, cnt, w_self, w_neigh, b)


def kernel(x, src1, dst1, src2, dst2, src3, dst3, n_dst1, n_dst2, n_dst3,
           W_self1, W_neigh1, b1, W_self2, W_neigh2, b2,
           W_self3, W_neigh3, b3):
    N1, N2, N3 = 20000, 8000, 4096
    n_src = x.shape[0]
    zero = ((jnp.asarray(n_dst1) - N1)
            + (jnp.asarray(n_dst2) - N2)
            + (jnp.asarray(n_dst3) - N3)).astype(x.dtype)

    agg1, cnt1 = _sc_mean_agg(x.reshape(2 * n_src, 64), src1, dst1, n_src, N1,
                              fh=64, chunk=80, sb=50, interleave=True)
    h1 = _tc_sage(x, agg1, cnt1, W_self1, W_neigh1,
                  b1.reshape(1, 256), relu=True, split_out=True,
                  n_out=N1, bm=800)

    agg2, cnt2 = _sc_mean_agg(h1.reshape(2 * N1, 128), src2, dst2, N1, N2,
                              fh=128, chunk=80, sb=100)
    h2 = _tc_sage(h1, agg2, cnt2, W_self2.reshape(2, 128, 256),
                  W_neigh2.reshape(2, 128, 256), b2.reshape(1, 256),
                  relu=True, split_out=True, n_out=N2, bm=800)

    agg3, cnt3 = _sc_mean_agg(h2.reshape(2 * N2, 128), src3, dst3, N2, N3,
                              fh=128, chunk=128, sb=32)
    h3 = _tc_sage(h2, agg3, cnt3, W_self3.reshape(2, 128, 256),
                  W_neigh3.reshape(2, 128, 256), (b3 + zero).reshape(1, 256),
                  relu=False, split_out=False, n_out=N3, bm=1024)
    return h3
